# baseline (speedup 1.0000x reference)
import functools

import jax
import jax.numpy as jnp
from jax import lax
from jax.experimental import pallas as pl
from jax.experimental.pallas import tpu as pltpu
from jax.experimental.pallas import tpu_sc as plsc

NFEAT = 128
D = 48
CH = 256
SCH = 128
NC = 2
NS = 16
NW = NC * NS
N_PAD = 10240
RPT = N_PAD // NS
NCH = RPT // SCH
NBUF = 8
EDIX = 48


def _fill_edix(edix, wid, per_w):
    for col in range(EDIX // 16):
        sl = pl.ds(col * 16, 16)
        k = col * 16 + lax.iota(jnp.int32, 16)
        edix[0, sl] = jnp.where(k < per_w, wid * per_w + k,
                                jnp.int32(NW * per_w - 1))


def _sc_degree(ed2):
    per_w = ed2.shape[0] // NW
    mesh = plsc.VectorSubcoreMesh(core_axis_name="c", subcore_axis_name="s")

    @functools.partial(
        pl.kernel,
        mesh=mesh,
        out_type=jax.ShapeDtypeStruct((NC, N_PAD), jnp.float32),
        compiler_params=pltpu.CompilerParams(use_tc_tiling_on_sc=False),
        scratch_types=[
            pltpu.VMEM((EDIX, CH), jnp.int32),
            pltpu.VMEM((per_w, CH), jnp.int32),
            pltpu.VMEM((CH,), jnp.float32),
            pltpu.VMEM((SCH,), jnp.float32),
            pltpu.VMEM((8, EDIX), jnp.int32),
            pltpu.VMEM_SHARED((N_PAD,), jnp.float32),
            pltpu.SemaphoreType.DMA,
        ],
    )
    def k(ed_hbm, out_hbm, ed_v, idx_v, ones_v, zbuf, edix, acc, dsem):
        c = lax.axis_index("c")
        s = lax.axis_index("s")
        wid = c * NS + s

        for r in range(CH // 16):
            ones_v[pl.ds(r * 16, 16)] = jnp.ones((16,), jnp.float32)
        for r in range(SCH // 16):
            zbuf[pl.ds(r * 16, 16)] = jnp.zeros((16,), jnp.float32)
        _fill_edix(edix, wid, per_w)

        base = s * RPT

        def zero_slab(i, carry):
            pltpu.sync_copy(zbuf, acc.at[pl.ds(base + i * SCH, SCH)])
            return carry

        lax.fori_loop(0, NCH, zero_slab, 0)

        pltpu.async_copy(ed_hbm.at[edix.at[0]], ed_v, dsem).wait()

        def unpack(r, carry):
            for col in range(CH // 16):
                sl = pl.ds(col * 16, 16)
                idx_v[r, sl] = lax.shift_right_logical(ed_v[r, sl],
                                                       jnp.int32(16))
            return carry

        lax.fori_loop(0, per_w, unpack, 0)
        plsc.subcore_barrier()

        def body(j, carry):
            pltpu.async_copy(ones_v, acc.at[idx_v.at[j]], dsem, add=True)
            return carry

        lax.fori_loop(0, per_w, body, 0)

        def drain(j, carry):
            pltpu.make_async_copy(ones_v, acc.at[idx_v.at[j]], dsem).wait()
            return carry

        lax.fori_loop(0, per_w, drain, 0)
        plsc.subcore_barrier()

        pltpu.sync_copy(acc.at[pl.ds(base, RPT)],
                        out_hbm.at[c, pl.ds(base, RPT)])

    return k(ed2)


def _sc_edges(ed2, h2):
    per_w = ed2.shape[0] // NW
    mesh = plsc.VectorSubcoreMesh(core_axis_name="c", subcore_axis_name="s")

    @functools.partial(
        pl.kernel,
        mesh=mesh,
        out_type=jax.ShapeDtypeStruct((NC, N_PAD, D), jnp.bfloat16),
        compiler_params=pltpu.CompilerParams(use_tc_tiling_on_sc=False),
        scratch_types=[
            pltpu.VMEM((EDIX, CH), jnp.int32),
            pltpu.VMEM((per_w, CH), jnp.int32),
            pltpu.VMEM((per_w, CH), jnp.int32),
            pltpu.VMEM((NBUF, CH, D), jnp.bfloat16),
            pltpu.VMEM((SCH, D), jnp.bfloat16),
            pltpu.VMEM((NCH, SCH), jnp.int32),
            pltpu.VMEM((8, EDIX), jnp.int32),
            pltpu.VMEM_SHARED((N_PAD, D), jnp.bfloat16),
            pltpu.VMEM_SHARED((N_PAD, D), jnp.bfloat16),
            pltpu.SemaphoreType.DMA((NBUF,)),
            pltpu.SemaphoreType.DMA((NBUF,)),
            pltpu.SemaphoreType.DMA,
        ],
    )
    def k(ed_hbm, h2_hbm, out_hbm, ed_v, src_v, dst_v, rows, sbuf, iost,
          edix, h2s, acc, gsem, ssem, xsem):
        c = lax.axis_index("c")
        s = lax.axis_index("s")
        wid = c * NS + s
        base = s * RPT

        def fill_idx(i, carry):
            for col in range(SCH // 16):
                sl = pl.ds(col * 16, 16)
                iost[i, sl] = (base + i * SCH + col * 16
                               + lax.iota(jnp.int32, 16))
            return carry

        lax.fori_loop(0, NCH, fill_idx, 0)
        _fill_edix(edix, wid, per_w)

        def stage_h2(i, carry):
            pltpu.async_copy(h2_hbm.at[iost.at[i]], sbuf, xsem).wait()
            pltpu.sync_copy(sbuf, h2s.at[pl.ds(base + i * SCH, SCH)])
            pltpu.sync_copy(sbuf, acc.at[pl.ds(base + i * SCH, SCH)])
            return carry

        lax.fori_loop(0, NCH, stage_h2, 0)

        pltpu.async_copy(ed_hbm.at[edix.at[0]], ed_v, xsem).wait()

        def unpack(r, carry):
            for col in range(CH // 16):
                sl = pl.ds(col * 16, 16)
                v = ed_v[r, sl]
                src_v[r, sl] = lax.bitwise_and(v, jnp.int32(0xFFFF))
                dst_v[r, sl] = lax.shift_right_logical(v, jnp.int32(16))
            return carry

        lax.fori_loop(0, per_w, unpack, 0)
        plsc.subcore_barrier()

        rounds = per_w // NBUF
        for b in range(NBUF):
            pltpu.async_copy(h2s.at[src_v.at[b]], rows.at[b], gsem.at[b])

        def round_body(r, carry):
            for b in range(NBUF):
                j = r * NBUF + b
                pltpu.make_async_copy(h2s.at[src_v.at[j]], rows.at[b],
                                      gsem.at[b]).wait()
                pltpu.async_copy(rows.at[b], acc.at[dst_v.at[j]], ssem.at[b],
                                 add=True)

            @pl.when(r < rounds - 1)
            def _issue_next():
                for b in range(NBUF):
                    jn = (r + 1) * NBUF + b
                    pltpu.make_async_copy(rows.at[b], acc.at[dst_v.at[jn]],
                                          ssem.at[b]).wait()
                    pltpu.async_copy(h2s.at[src_v.at[jn]], rows.at[b],
                                     gsem.at[b])

            return carry

        lax.fori_loop(0, rounds, round_body, 0)
        for b in range(NBUF):
            pltpu.make_async_copy(rows.at[b], acc.at[dst_v.at[b]],
                                  ssem.at[b]).wait()
        plsc.subcore_barrier()

        def dump(i, carry):
            pltpu.sync_copy(acc.at[pl.ds(base + i * SCH, SCH)], sbuf)
            pltpu.async_copy(sbuf, out_hbm.at[c].at[iost.at[i]], xsem).wait()
            return carry

        lax.fori_loop(0, NCH, dump, 0)

    return k(ed2, h2)


def _tc_matmul(x_pad, W_gcn, fcWp):
    def body(x_ref, w_ref, f_ref, g_ref):
        wc = lax.dot_general(w_ref[:], f_ref[:], (((1,), (1,)), ((), ())),
                             preferred_element_type=jnp.float32)
        g_ref[:] = lax.dot_general(x_ref[:], wc, (((1,), (0,)), ((), ())),
                                   preferred_element_type=jnp.float32)

    return pl.pallas_call(
        body,
        out_shape=jax.ShapeDtypeStruct((N_PAD, D), jnp.float32),
    )(x_pad, W_gcn, fcWp)


def _tc_scale(g, degp):
    def body(g_ref, deg_ref, h2_ref):
        deg = deg_ref[0] + deg_ref[1] + 1.0
        dinv = lax.rsqrt(deg)
        h2_ref[:] = (g_ref[:] * dinv).astype(jnp.bfloat16)

    return pl.pallas_call(
        body,
        out_shape=jax.ShapeDtypeStruct((N_PAD, D), jnp.bfloat16),
    )(g, degp)


def _tc_finalize(partials, h2, degp, b2, fcWp, fcb2):
    def body(p_ref, h2_ref, deg_ref, b_ref, f_ref, fb_ref, y_ref):
        deg = deg_ref[0] + deg_ref[1] + 1.0
        dinv = lax.rsqrt(deg)
        tot = (p_ref[0].astype(jnp.float32) + p_ref[1].astype(jnp.float32)
               - h2_ref[:].astype(jnp.float32))
        const = lax.dot_general(b_ref[:], f_ref[:], (((1,), (1,)), ((), ())),
                                preferred_element_type=jnp.float32) + fb_ref[:]
        y_ref[:] = tot * dinv + const

    return pl.pallas_call(
        body,
        out_shape=jax.ShapeDtypeStruct((N_PAD, D), jnp.float32),
    )(partials, h2, degp, b2, fcWp, fcb2)


def kernel(x, edge_index, W_gcn, b_gcn, fc_W, fc_b):
    N = x.shape[0]
    nclass = fc_W.shape[0]
    src = edge_index[0].astype(jnp.int32)
    dst = edge_index[1].astype(jnp.int32)
    E = src.shape[0]
    per_w = -(-E // (NW * CH))
    per_w = -(-per_w // NBUF) * NBUF
    e_pad = NW * per_w * CH
    packed = jnp.bitwise_or(src, jnp.left_shift(dst, 16))
    pad_val = jnp.int32(N | (N << 16))
    ed2 = jnp.full((e_pad,), pad_val, jnp.int32).at[:E].set(packed)
    ed2 = ed2.reshape(NW * per_w, CH)
    x_pad = jnp.zeros((N_PAD, NFEAT), x.dtype).at[:N].set(x)
    fcWp = jnp.zeros((D, NFEAT), fc_W.dtype).at[:nclass].set(fc_W)
    fcb2 = jnp.zeros((1, D), fc_b.dtype).at[0, :nclass].set(fc_b)
    b2 = b_gcn.reshape(1, NFEAT)

    g = _tc_matmul(x_pad, W_gcn, fcWp)
    degp = _sc_degree(ed2).reshape(NC, N_PAD, 1)
    h2 = _tc_scale(g, degp)
    partials = _sc_edges(ed2, h2)
    y48 = _tc_finalize(partials, h2, degp, b2, fcWp, fcb2)
    return y48[:N, :nclass]


# speedup vs baseline: 1.1482x; 1.1482x over previous
import functools

import jax
import jax.numpy as jnp
from jax import lax
from jax.experimental import pallas as pl
from jax.experimental.pallas import tpu as pltpu
from jax.experimental.pallas import tpu_sc as plsc

NFEAT = 128
D = 48
CH = 128
NC = 2
NS = 16
NW = NC * NS
N_PAD = 10240
RPT = N_PAD // NS
NCH = RPT // CH
DEGW = 16
NBUF = 8


def _sc_degree(ed2):
    per_w = ed2.shape[0] // NW
    mesh = plsc.VectorSubcoreMesh(core_axis_name="c", subcore_axis_name="s")

    @functools.partial(
        pl.kernel,
        mesh=mesh,
        out_type=jax.ShapeDtypeStruct((NC, N_PAD), jnp.float32),
        compiler_params=pltpu.CompilerParams(use_tc_tiling_on_sc=False),
        scratch_types=[
            pltpu.VMEM((per_w, CH), jnp.int32),
            pltpu.VMEM((per_w, CH), jnp.int32),
            pltpu.VMEM((CH,), jnp.float32),
            pltpu.VMEM((CH,), jnp.float32),
            pltpu.VMEM((8, per_w), jnp.int32),
            pltpu.VMEM_SHARED((N_PAD,), jnp.float32),
            pltpu.SemaphoreType.DMA,
        ],
    )
    def k(ed_hbm, out_hbm, ed_v, idx_v, ones_v, zbuf, edix, acc, dsem):
        c = lax.axis_index("c")
        s = lax.axis_index("s")
        wid = c * NS + s

        for r in range(CH // 16):
            sl = pl.ds(r * 16, 16)
            ones_v[sl] = jnp.ones((16,), jnp.float32)
            zbuf[sl] = jnp.zeros((16,), jnp.float32)

        for col in range(per_w // 16):
            sl = pl.ds(col * 16, 16)
            edix[0, sl] = wid * per_w + col * 16 + lax.iota(jnp.int32, 16)

        base = s * RPT

        def zero_slab(i, carry):
            pltpu.sync_copy(zbuf, acc.at[pl.ds(base + i * CH, CH)])
            return carry

        lax.fori_loop(0, NCH, zero_slab, 0)

        pltpu.async_copy(ed_hbm.at[edix.at[0]], ed_v, dsem).wait()

        def unpack(r, carry):
            for col in range(CH // 16):
                sl = pl.ds(col * 16, 16)
                idx_v[r, sl] = lax.shift_right_logical(ed_v[r, sl],
                                                       jnp.int32(16))
            return carry

        lax.fori_loop(0, per_w, unpack, 0)
        plsc.subcore_barrier()

        def body(j, carry):
            pltpu.async_copy(ones_v, acc.at[idx_v.at[j]], dsem, add=True)
            return carry

        lax.fori_loop(0, per_w, body, 0)

        def drain(j, carry):
            pltpu.make_async_copy(ones_v, acc.at[idx_v.at[j]], dsem).wait()
            return carry

        lax.fori_loop(0, per_w, drain, 0)
        plsc.subcore_barrier()

        pltpu.sync_copy(acc.at[pl.ds(base, RPT)],
                        out_hbm.at[c, pl.ds(base, RPT)])

    return k(ed2)


def _sc_edges(ed2, h2):
    per_w = ed2.shape[0] // NW
    mesh = plsc.VectorSubcoreMesh(core_axis_name="c", subcore_axis_name="s")

    @functools.partial(
        pl.kernel,
        mesh=mesh,
        out_type=jax.ShapeDtypeStruct((NC, N_PAD, D), jnp.bfloat16),
        compiler_params=pltpu.CompilerParams(use_tc_tiling_on_sc=False),
        scratch_types=[
            pltpu.VMEM((per_w, CH), jnp.int32),
            pltpu.VMEM((per_w, CH), jnp.int32),
            pltpu.VMEM((per_w, CH), jnp.int32),
            pltpu.VMEM((NBUF, CH, D), jnp.bfloat16),
            pltpu.VMEM((NCH, CH), jnp.int32),
            pltpu.VMEM((8, per_w), jnp.int32),
            pltpu.VMEM_SHARED((N_PAD, D), jnp.bfloat16),
            pltpu.VMEM_SHARED((N_PAD, D), jnp.bfloat16),
            pltpu.SemaphoreType.DMA((NBUF,)),
            pltpu.SemaphoreType.DMA((NBUF,)),
            pltpu.SemaphoreType.DMA,
        ],
    )
    def k(ed_hbm, h2_hbm, out_hbm, ed_v, src_v, dst_v, rows, iost,
          edix, h2s, acc, gsem, ssem, xsem):
        c = lax.axis_index("c")
        s = lax.axis_index("s")
        wid = c * NS + s
        base = s * RPT

        def fill_idx(i, carry):
            for col in range(CH // 16):
                sl = pl.ds(col * 16, 16)
                v = base + i * CH + col * 16 + lax.iota(jnp.int32, 16)
                iost[i, sl] = v
            return carry

        lax.fori_loop(0, NCH, fill_idx, 0)
        for col in range(per_w // 16):
            sl = pl.ds(col * 16, 16)
            edix[0, sl] = wid * per_w + col * 16 + lax.iota(jnp.int32, 16)

        def stage_h2(i, carry):
            pltpu.async_copy(h2_hbm.at[iost.at[i]], rows.at[0], xsem).wait()
            pltpu.sync_copy(rows.at[0], h2s.at[pl.ds(base + i * CH, CH)])
            pltpu.sync_copy(rows.at[0], acc.at[pl.ds(base + i * CH, CH)])
            return carry

        lax.fori_loop(0, NCH, stage_h2, 0)

        pltpu.async_copy(ed_hbm.at[edix.at[0]], ed_v, xsem).wait()

        def unpack(r, carry):
            for col in range(CH // 16):
                sl = pl.ds(col * 16, 16)
                v = ed_v[r, sl]
                src_v[r, sl] = lax.bitwise_and(v, jnp.int32(0xFFFF))
                dst_v[r, sl] = lax.shift_right_logical(v, jnp.int32(16))
            return carry

        lax.fori_loop(0, per_w, unpack, 0)
        plsc.subcore_barrier()

        def gsrc(b):
            return h2s

        rounds = per_w // NBUF
        for b in range(NBUF):
            pltpu.async_copy(gsrc(b).at[src_v.at[b]], rows.at[b], gsem.at[b])

        def round_body(r, carry):
            for b in range(NBUF):
                j = r * NBUF + b
                pltpu.make_async_copy(gsrc(b).at[src_v.at[j]], rows.at[b],
                                      gsem.at[b]).wait()
                pltpu.async_copy(rows.at[b], acc.at[dst_v.at[j]], ssem.at[b],
                                 add=True)

            @pl.when(r < rounds - 1)
            def _issue_next():
                for b in range(NBUF):
                    jn = (r + 1) * NBUF + b
                    pltpu.make_async_copy(rows.at[b], acc.at[dst_v.at[jn]],
                                          ssem.at[b]).wait()
                    pltpu.async_copy(gsrc(b).at[src_v.at[jn]], rows.at[b],
                                     gsem.at[b])

            return carry

        lax.fori_loop(0, rounds, round_body, 0)
        for b in range(NBUF):
            pltpu.make_async_copy(rows.at[b], acc.at[dst_v.at[b]],
                                  ssem.at[b]).wait()
        plsc.subcore_barrier()

        def dump(i, carry):
            pltpu.sync_copy(acc.at[pl.ds(base + i * CH, CH)], rows.at[0])
            pltpu.async_copy(rows.at[0], out_hbm.at[c].at[iost.at[i]],
                             xsem).wait()
            return carry

        lax.fori_loop(0, NCH, dump, 0)

    return k(ed2, h2)


def _tc_matmul(x_pad, W_gcn, fcWp):
    def body(x_ref, w_ref, f_ref, g_ref):
        wc = lax.dot_general(w_ref[:], f_ref[:], (((1,), (1,)), ((), ())),
                             preferred_element_type=jnp.float32)
        g_ref[:] = lax.dot_general(x_ref[:], wc, (((1,), (0,)), ((), ())),
                                   preferred_element_type=jnp.float32)

    return pl.pallas_call(
        body,
        out_shape=jax.ShapeDtypeStruct((N_PAD, D), jnp.float32),
    )(x_pad, W_gcn, fcWp)


def _tc_scale(g, degp):
    def body(g_ref, deg_ref, h2_ref):
        deg = deg_ref[0] + deg_ref[1] + 1.0
        dinv = lax.rsqrt(deg)
        h2_ref[:] = (g_ref[:] * dinv).astype(jnp.bfloat16)

    return pl.pallas_call(
        body,
        out_shape=jax.ShapeDtypeStruct((N_PAD, D), jnp.bfloat16),
    )(g, degp)


def _tc_finalize(partials, h2, degp, b2, fcWp, fcb2):
    def body(p_ref, h2_ref, deg_ref, b_ref, f_ref, fb_ref, y_ref):
        deg = deg_ref[0] + deg_ref[1] + 1.0
        dinv = lax.rsqrt(deg)
        tot = (p_ref[0].astype(jnp.float32) + p_ref[1].astype(jnp.float32)
               - h2_ref[:].astype(jnp.float32))
        const = lax.dot_general(b_ref[:], f_ref[:], (((1,), (1,)), ((), ())),
                                preferred_element_type=jnp.float32) + fb_ref[:]
        y_ref[:] = tot * dinv + const

    return pl.pallas_call(
        body,
        out_shape=jax.ShapeDtypeStruct((N_PAD, D), jnp.float32),
    )(partials, h2, degp, b2, fcWp, fcb2)


def kernel(x, edge_index, W_gcn, b_gcn, fc_W, fc_b):
    N = x.shape[0]
    nclass = fc_W.shape[0]
    src = edge_index[0].astype(jnp.int32)
    dst = edge_index[1].astype(jnp.int32)
    E = src.shape[0]
    per_w = -(-E // (NW * CH))
    per_w = -(-per_w // NBUF) * NBUF
    e_pad = NW * per_w * CH
    packed = jnp.bitwise_or(src, jnp.left_shift(dst, 16))
    pad_val = jnp.int32(N | (N << 16))
    ed2 = jnp.full((e_pad,), pad_val, jnp.int32).at[:E].set(packed)
    ed2 = ed2.reshape(NW * per_w, CH)
    x_pad = jnp.zeros((N_PAD, NFEAT), x.dtype).at[:N].set(x)
    fcWp = jnp.zeros((D, NFEAT), fc_W.dtype).at[:nclass].set(fc_W)
    fcb2 = jnp.zeros((1, D), fc_b.dtype).at[0, :nclass].set(fc_b)
    b2 = b_gcn.reshape(1, NFEAT)

    g = _tc_matmul(x_pad, W_gcn, fcWp)
    degp = _sc_degree(ed2).reshape(NC, N_PAD, 1)
    h2 = _tc_scale(g, degp)
    partials = _sc_edges(ed2, h2)
    y48 = _tc_finalize(partials, h2, degp, b2, fcWp, fcb2)
    return y48[:N, :nclass]
